# baseline (speedup 1.0000x reference)
import functools

import jax
import jax.numpy as jnp
from jax import lax
from jax.experimental import pallas as pl
from jax.experimental.pallas import tpu as pltpu
from jax.experimental.pallas import tpu_sc as plsc

NODE_DIM = 512
BATCH = 16
TOTAL = 32768
BLK = 8192
NBLK = TOTAL // BLK

_NC = 2
_NS = 16
_NW = _NC * _NS
_CHUNK = 64
_PER_W = TOTAL // _NW
_NCHUNK = _PER_W // _CHUNK
_ACC_ROWS = BATCH + 1


def _seg_sums_sc(x, cum):
    mesh = plsc.VectorSubcoreMesh(core_axis_name="c", subcore_axis_name="s")
    zeros = jnp.zeros((_ACC_ROWS * NODE_DIM,), jnp.float32)

    @functools.partial(
        pl.kernel, mesh=mesh,
        out_type=jax.ShapeDtypeStruct((_NW, BATCH * NODE_DIM), jnp.float32),
        scratch_types=[
            pltpu.VMEM((_CHUNK, NODE_DIM), jnp.float32),
            pltpu.VMEM((_CHUNK, NODE_DIM), jnp.float32),
            pltpu.VMEM((BATCH,), jnp.int32),
            pltpu.VMEM((_ACC_ROWS * NODE_DIM,), jnp.float32),
            pltpu.SemaphoreType.DMA,
            pltpu.SemaphoreType.DMA,
        ],
    )
    def sc_fn(x_hbm, cref, zz, out, buf0, buf1, cntv, acc, sem0, sem1):
        cid = lax.axis_index("c")
        sid = lax.axis_index("s")
        wid = sid * _NC + cid
        base = wid * _PER_W

        pltpu.sync_copy(zz, acc)
        pltpu.sync_copy(cref, cntv)
        cv = cntv[...]
        cums = [cv[g] for g in range(BATCH)]

        def start(k, buf, sem):
            pltpu.make_async_copy(
                x_hbm.at[pl.ds(base + k * _CHUNK, _CHUNK)], buf, sem).start()

        def wait(k, buf, sem):
            pltpu.make_async_copy(
                x_hbm.at[pl.ds(base + k * _CHUNK, _CHUNK)], buf, sem).wait()

        def process(k, buf):
            @plsc.parallel_loop(0, _CHUNK // 16)
            def group_body(q, buf=buf):
                tq = base + k * _CHUNK + q * 16
                tv = lax.iota(jnp.int32, 16) + tq
                neg = jnp.zeros((16,), jnp.int32)
                for g in range(BATCH):
                    neg += lax.shift_right_logical(tv - cums[g], 31)
                sv = (16 - neg) * NODE_DIM

                @pl.when(sv[0] == sv[15])
                def _():
                    soff = sv[0]
                    for c in range(NODE_DIM // 16):
                        v = [buf[q * 16 + j, pl.ds(c * 16, 16)]
                             for j in range(16)]
                        while len(v) > 1:
                            v = [v[2 * m] + v[2 * m + 1]
                                 for m in range(len(v) // 2)]
                        plsc.addupdate(acc.at[pl.ds(soff + c * 16, 16)], v[0])

                @pl.when(sv[0] != sv[15])
                def _():
                    for j in range(16):
                        soff = sv[j]
                        for c in range(NODE_DIM // 16):
                            plsc.addupdate(
                                acc.at[pl.ds(soff + c * 16, 16)],
                                buf[q * 16 + j, pl.ds(c * 16, 16)])

        start(0, buf0, sem0)

        def pair_body(m, carry):
            k0 = 2 * m
            wait(k0, buf0, sem0)

            @pl.when(k0 + 1 < _NCHUNK)
            def _():
                start(k0 + 1, buf1, sem1)

            process(k0, buf0)

            @pl.when(k0 + 2 < _NCHUNK)
            def _():
                start(k0 + 2, buf0, sem0)

            @pl.when(k0 + 1 < _NCHUNK)
            def _():
                wait(k0 + 1, buf1, sem1)
                process(k0 + 1, buf1)
            return carry

        lax.fori_loop(0, (_NCHUNK + 1) // 2, pair_body, 0)
        pltpu.sync_copy(acc.at[pl.ds(0, BATCH * NODE_DIM)], out.at[wid])

    return sc_fn(x, cum, zeros)


def _onehot(pos, s_ref, c_ref):
    s = s_ref[...]
    c = c_ref[...]
    return ((pos >= s) & (pos < s + c)).astype(jnp.float32)


def _pass1_body(xn_ref, sn_ref, cn_ref, outn_ref, accn):
    i = pl.program_id(0)
    pos = jax.lax.broadcasted_iota(jnp.int32, (BATCH, BLK), 1) + i * BLK
    mn = _onehot(pos, sn_ref, cn_ref)
    dn = (((1,), (0,)), ((), ()))
    pn = jax.lax.dot_general(mn, xn_ref[...], dn,
                             precision=jax.lax.Precision.DEFAULT,
                             preferred_element_type=jnp.float32)

    @pl.when(i == 0)
    def _():
        accn[...] = pn

    @pl.when(i > 0)
    def _():
        accn[...] += pn

    @pl.when(i == pl.num_programs(0) - 1)
    def _():
        outn_ref[...] = accn[...]


def _pass2_body(reduce_sums, sum_ref, th_ref, sn_ref, cn_ref,
                x_ref, out_ref, acc, com):
    i = pl.program_id(0)
    dnums_t = (((1,), (1,)), ((), ()))
    dnums_m = (((1,), (0,)), ((), ()))

    @pl.when(i == 0)
    def _():
        den = jnp.maximum(cn_ref[...].astype(jnp.float32), 1.0)
        s = jnp.sum(sum_ref[...], axis=0) if reduce_sums else sum_ref[...]
        com[...] = jnp.maximum(
            jax.lax.dot_general(s / den, th_ref[...], dnums_t,
                                preferred_element_type=jnp.float32), 0.0)
        acc[...] = jnp.zeros_like(acc)

    @pl.when(i > 0)
    def _():
        pos = jax.lax.broadcasted_iota(jnp.int32, (BATCH, BLK), 1) + (i - 1) * BLK
        mn = _onehot(pos, sn_ref, cn_ref)
        xn = x_ref[...]
        s16 = jax.lax.dot_general(com[...], xn, dnums_t,
                                  precision=jax.lax.Precision.DEFAULT,
                                  preferred_element_type=jnp.float32)
        gn = mn / (1.0 + jnp.exp(-s16))
        acc[...] += jax.lax.dot_general(gn, xn, dnums_m,
                                        precision=jax.lax.Precision.DEFAULT,
                                        preferred_element_type=jnp.float32)

    @pl.when(i == pl.num_programs(0) - 1)
    def _():
        den = jnp.maximum(cn_ref[...].astype(jnp.float32), 1.0)
        out_ref[...] = acc[...] / den


def _pass2(sums, theta, starts, counts, x, reduce_sums):
    small = pl.BlockSpec((BATCH, 1), lambda i: (0, 0))
    full = lambda shp: pl.BlockSpec(shp, lambda i: (0,) * len(shp))
    xspec2 = pl.BlockSpec((BLK, NODE_DIM), lambda i: (jnp.maximum(i - 1, 0), 0))
    return pl.pallas_call(
        functools.partial(_pass2_body, reduce_sums),
        grid=(NBLK + 1,),
        in_specs=[full(sums.shape), full((NODE_DIM, NODE_DIM)),
                  small, small, xspec2],
        out_specs=full((BATCH, NODE_DIM)),
        out_shape=jax.ShapeDtypeStruct((BATCH, NODE_DIM), jnp.float32),
        scratch_shapes=[pltpu.VMEM((BATCH, NODE_DIM), jnp.float32)] * 2,
        compiler_params=pltpu.CompilerParams(
            dimension_semantics=("arbitrary",)),
    )(sums, theta, starts, counts, x)


def kernel(eb_nodes, eb_edges, numb_nodes, numb_edges, theta_obj, theta_pred):
    cum_n = jnp.cumsum(numb_nodes).astype(jnp.int32)
    cum_e = jnp.cumsum(numb_edges).astype(jnp.int32)
    starts_n = (cum_n - numb_nodes).reshape(BATCH, 1)
    starts_e = (cum_e - numb_edges).reshape(BATCH, 1)
    counts_n = numb_nodes.reshape(BATCH, 1)
    counts_e = numb_edges.reshape(BATCH, 1)

    sums_e = _seg_sums_sc(eb_edges, cum_e)
    sums_e = sums_e.reshape(_NW, BATCH, NODE_DIM)

    small = pl.BlockSpec((BATCH, 1), lambda i: (0, 0))
    full = lambda shp: pl.BlockSpec(shp, lambda i: (0,) * len(shp))
    xspec1 = pl.BlockSpec((BLK, NODE_DIM), lambda i: (i, 0))

    sums_n = pl.pallas_call(
        _pass1_body,
        grid=(NBLK,),
        in_specs=[xspec1, small, small],
        out_specs=full((BATCH, NODE_DIM)),
        out_shape=jax.ShapeDtypeStruct((BATCH, NODE_DIM), jnp.float32),
        scratch_shapes=[pltpu.VMEM((BATCH, NODE_DIM), jnp.float32)],
        compiler_params=pltpu.CompilerParams(
            dimension_semantics=("arbitrary",)),
    )(eb_nodes, starts_n, counts_n)

    out_n = _pass2(sums_n, theta_obj, starts_n, counts_n, eb_nodes,
                   reduce_sums=False)
    out_e = _pass2(sums_e, theta_pred, starts_e, counts_e, eb_edges,
                   reduce_sums=True)
    return jnp.concatenate([out_n, out_e], axis=1)


# speedup vs baseline: 1.0111x; 1.0111x over previous
import functools

import jax
import jax.numpy as jnp
from jax import lax
from jax.experimental import pallas as pl
from jax.experimental.pallas import tpu as pltpu
from jax.experimental.pallas import tpu_sc as plsc

NODE_DIM = 512
BATCH = 16
TOTAL = 32768
BLK = 4096
NBLK = TOTAL // BLK

_NC = 2
_NS = 16
_NW = _NC * _NS
_CHUNK = 64
_SC_TOKENS = TOTAL // 2
_PER_W = _SC_TOKENS // _NW
_NCHUNK = _PER_W // _CHUNK
_ACC_ROWS = BATCH + 1


def _seg_sums_sc(x, cum):
    mesh = plsc.VectorSubcoreMesh(core_axis_name="c", subcore_axis_name="s")
    zeros = jnp.zeros((_ACC_ROWS * NODE_DIM,), jnp.float32)

    @functools.partial(
        pl.kernel, mesh=mesh,
        out_type=jax.ShapeDtypeStruct((_NW, BATCH * NODE_DIM), jnp.float32),
        scratch_types=[
            pltpu.VMEM((_CHUNK, NODE_DIM), jnp.float32),
            pltpu.VMEM((_CHUNK, NODE_DIM), jnp.float32),
            pltpu.VMEM((BATCH,), jnp.int32),
            pltpu.VMEM((_ACC_ROWS * NODE_DIM,), jnp.float32),
            pltpu.SemaphoreType.DMA,
            pltpu.SemaphoreType.DMA,
        ],
    )
    def sc_fn(x_hbm, cref, zz, out, buf0, buf1, cntv, acc, sem0, sem1):
        cid = lax.axis_index("c")
        sid = lax.axis_index("s")
        wid = sid * _NC + cid
        base = wid * _PER_W

        pltpu.sync_copy(zz, acc)
        pltpu.sync_copy(cref, cntv)
        cv = cntv[...]
        cums = [cv[g] for g in range(BATCH)]

        def start(k, buf, sem):
            pltpu.make_async_copy(
                x_hbm.at[pl.ds(base + k * _CHUNK, _CHUNK)], buf, sem).start()

        def wait(k, buf, sem):
            pltpu.make_async_copy(
                x_hbm.at[pl.ds(base + k * _CHUNK, _CHUNK)], buf, sem).wait()

        def process(k, buf):
            @plsc.parallel_loop(0, _CHUNK // 16)
            def group_body(q, buf=buf):
                tq = base + k * _CHUNK + q * 16
                tv = lax.iota(jnp.int32, 16) + tq
                neg = jnp.zeros((16,), jnp.int32)
                for g in range(BATCH):
                    neg += lax.shift_right_logical(tv - cums[g], 31)
                sv = (16 - neg) * NODE_DIM

                @pl.when(sv[0] == sv[15])
                def _():
                    soff = sv[0]
                    for c in range(NODE_DIM // 16):
                        v = [buf[q * 16 + j, pl.ds(c * 16, 16)]
                             for j in range(16)]
                        while len(v) > 1:
                            v = [v[2 * m] + v[2 * m + 1]
                                 for m in range(len(v) // 2)]
                        plsc.addupdate(acc.at[pl.ds(soff + c * 16, 16)], v[0])

                @pl.when(sv[0] != sv[15])
                def _():
                    for j in range(16):
                        soff = sv[j]
                        for c in range(NODE_DIM // 16):
                            plsc.addupdate(
                                acc.at[pl.ds(soff + c * 16, 16)],
                                buf[q * 16 + j, pl.ds(c * 16, 16)])

        start(0, buf0, sem0)

        def pair_body(m, carry):
            k0 = 2 * m
            wait(k0, buf0, sem0)

            @pl.when(k0 + 1 < _NCHUNK)
            def _():
                start(k0 + 1, buf1, sem1)

            process(k0, buf0)

            @pl.when(k0 + 2 < _NCHUNK)
            def _():
                start(k0 + 2, buf0, sem0)

            @pl.when(k0 + 1 < _NCHUNK)
            def _():
                wait(k0 + 1, buf1, sem1)
                process(k0 + 1, buf1)
            return carry

        lax.fori_loop(0, (_NCHUNK + 1) // 2, pair_body, 0)
        pltpu.sync_copy(acc.at[pl.ds(0, BATCH * NODE_DIM)], out.at[wid])

    return sc_fn(x, cum, zeros)


def _onehot(pos, s_ref, c_ref):
    s = s_ref[...]
    c = c_ref[...]
    return ((pos >= s) & (pos < s + c)).astype(jnp.float32)


def _pass1_body(off, xn_ref, sn_ref, cn_ref, outn_ref, accn):
    i = pl.program_id(0)
    pos = jax.lax.broadcasted_iota(jnp.int32, (BATCH, BLK), 1) + off + i * BLK
    mn = _onehot(pos, sn_ref, cn_ref)
    dn = (((1,), (0,)), ((), ()))
    pn = jax.lax.dot_general(mn, xn_ref[...], dn,
                             precision=jax.lax.Precision.DEFAULT,
                             preferred_element_type=jnp.float32)

    @pl.when(i == 0)
    def _():
        accn[...] = pn

    @pl.when(i > 0)
    def _():
        accn[...] += pn

    @pl.when(i == pl.num_programs(0) - 1)
    def _():
        outn_ref[...] = accn[...]


def _pass2_body(reduce_sums, sum_ref, th_ref, sn_ref, cn_ref,
                x_ref, out_ref, acc, com):
    i = pl.program_id(0)
    dnums_t = (((1,), (1,)), ((), ()))
    dnums_m = (((1,), (0,)), ((), ()))

    @pl.when(i == 0)
    def _():
        den = jnp.maximum(cn_ref[...].astype(jnp.float32), 1.0)
        s = jnp.sum(sum_ref[...], axis=0) if reduce_sums else sum_ref[...]
        com[...] = jnp.maximum(
            jax.lax.dot_general(s / den, th_ref[...], dnums_t,
                                preferred_element_type=jnp.float32), 0.0)
        acc[...] = jnp.zeros_like(acc)

    @pl.when(i > 0)
    def _():
        pos = jax.lax.broadcasted_iota(jnp.int32, (BATCH, BLK), 1) + (i - 1) * BLK
        mn = _onehot(pos, sn_ref, cn_ref)
        xn = x_ref[...]
        s16 = jax.lax.dot_general(com[...], xn, dnums_t,
                                  precision=jax.lax.Precision.DEFAULT,
                                  preferred_element_type=jnp.float32)
        gn = mn / (1.0 + jnp.exp(-s16))
        acc[...] += jax.lax.dot_general(gn, xn, dnums_m,
                                        precision=jax.lax.Precision.DEFAULT,
                                        preferred_element_type=jnp.float32)

    @pl.when(i == pl.num_programs(0) - 1)
    def _():
        den = jnp.maximum(cn_ref[...].astype(jnp.float32), 1.0)
        out_ref[...] = acc[...] / den


def _pass2(sums, theta, starts, counts, x, reduce_sums):
    small = pl.BlockSpec((BATCH, 1), lambda i: (0, 0))
    full = lambda shp: pl.BlockSpec(shp, lambda i: (0,) * len(shp))
    xspec2 = pl.BlockSpec((BLK, NODE_DIM), lambda i: (jnp.maximum(i - 1, 0), 0))
    return pl.pallas_call(
        functools.partial(_pass2_body, reduce_sums),
        grid=(NBLK + 1,),
        in_specs=[full(sums.shape), full((NODE_DIM, NODE_DIM)),
                  small, small, xspec2],
        out_specs=full((BATCH, NODE_DIM)),
        out_shape=jax.ShapeDtypeStruct((BATCH, NODE_DIM), jnp.float32),
        scratch_shapes=[pltpu.VMEM((BATCH, NODE_DIM), jnp.float32)] * 2,
        compiler_params=pltpu.CompilerParams(
            dimension_semantics=("arbitrary",)),
    )(sums, theta, starts, counts, x)


def kernel(eb_nodes, eb_edges, numb_nodes, numb_edges, theta_obj, theta_pred):
    cum_n = jnp.cumsum(numb_nodes).astype(jnp.int32)
    cum_e = jnp.cumsum(numb_edges).astype(jnp.int32)
    starts_n = (cum_n - numb_nodes).reshape(BATCH, 1)
    starts_e = (cum_e - numb_edges).reshape(BATCH, 1)
    counts_n = numb_nodes.reshape(BATCH, 1)
    counts_e = numb_edges.reshape(BATCH, 1)

    sums_e_sc = _seg_sums_sc(eb_edges, cum_e)
    sums_e_sc = sums_e_sc.reshape(_NW, BATCH, NODE_DIM)

    small = pl.BlockSpec((BATCH, 1), lambda i: (0, 0))
    full = lambda shp: pl.BlockSpec(shp, lambda i: (0,) * len(shp))
    xspec1 = pl.BlockSpec((BLK, NODE_DIM), lambda i: (i, 0))

    def _pass1_tc(x, starts, counts, off, nblk, xspec):
        return pl.pallas_call(
            functools.partial(_pass1_body, off),
            grid=(nblk,),
            in_specs=[xspec, small, small],
            out_specs=full((BATCH, NODE_DIM)),
            out_shape=jax.ShapeDtypeStruct((BATCH, NODE_DIM), jnp.float32),
            scratch_shapes=[pltpu.VMEM((BATCH, NODE_DIM), jnp.float32)],
            compiler_params=pltpu.CompilerParams(
                dimension_semantics=("arbitrary",)),
        )(x, starts, counts)

    sums_n = _pass1_tc(eb_nodes, starts_n, counts_n, 0, NBLK, xspec1)
    xspec1u = pl.BlockSpec((BLK, NODE_DIM),
                           lambda i: (i + _SC_TOKENS // BLK, 0))
    sums_e_tc = _pass1_tc(eb_edges, starts_e, counts_e, _SC_TOKENS,
                          (TOTAL - _SC_TOKENS) // BLK, xspec1u)

    sums_e = jnp.concatenate([sums_e_sc, sums_e_tc[None]], axis=0)
    out_n = _pass2(sums_n, theta_obj, starts_n, counts_n, eb_nodes,
                   reduce_sums=False)
    out_e = _pass2(sums_e, theta_pred, starts_e, counts_e, eb_edges,
                   reduce_sums=True)
    return jnp.concatenate([out_n, out_e], axis=1)
